# Initial kernel scaffold; baseline (speedup 1.0000x reference)
import jax
import jax.numpy as jnp
from jax.experimental import pallas as pl
from jax.experimental.pallas import tpu as pltpu
from jax.experimental.pallas import tpu_sc as plsc

_WIN = 128


def kernel(x, W):
    B, H = x.shape
    V, D = W.shape
    N = B * H

    idx = x.reshape(1, N)
    mesh = plsc.VectorSubcoreMesh(core_axis_name="core",
                                  subcore_axis_name="subcore")

    @pl.kernel(out_type=jax.ShapeDtypeStruct((N, D), W.dtype), mesh=mesh)
    def gather_kernel(w_hbm, i_hbm, o_hbm):
        def body(i_vmem, o_vmem):
            pltpu.sync_copy(w_hbm.at[i_vmem.at[0]], o_vmem)

        pltpu.emit_pipeline(
            body,
            grid=(N // _WIN,),
            in_specs=[pl.BlockSpec((1, _WIN), index_map=lambda i: (0, i))],
            out_specs=[pl.BlockSpec((_WIN, D), index_map=lambda i: (i, 0))],
            core_axis_name=("core", "subcore"),
            dimension_semantics=(pltpu.PARALLEL,),
        )(i_hbm, o_hbm)

    return gather_kernel(W, idx).reshape(B, H, D)


# speedup vs baseline: 1.7445x; 1.7445x over previous
import jax
import jax.numpy as jnp
from jax.experimental import pallas as pl
from jax.experimental.pallas import tpu as pltpu
from jax.experimental.pallas import tpu_sc as plsc

_WIN = 128


def kernel(x, W):
    B, H = x.shape
    V, D = W.shape
    N = B * H

    idx = x.reshape(1, N)
    mesh = plsc.VectorSubcoreMesh(core_axis_name="core",
                                  subcore_axis_name="subcore")

    @pl.kernel(out_type=jax.ShapeDtypeStruct((N, D), W.dtype), mesh=mesh,
               compiler_params=pltpu.CompilerParams(use_tc_tiling_on_sc=False))
    def gather_kernel(w_hbm, i_hbm, o_hbm):
        def body(i_vmem, o_vmem):
            pltpu.sync_copy(w_hbm.at[i_vmem.at[0]], o_vmem)

        pltpu.emit_pipeline(
            body,
            grid=(N // _WIN,),
            in_specs=[pl.BlockSpec((1, _WIN), index_map=lambda i: (0, i))],
            out_specs=[pl.BlockSpec((_WIN, D), index_map=lambda i: (i, 0))],
            core_axis_name=("core", "subcore"),
            dimension_semantics=(pltpu.PARALLEL,),
        )(i_hbm, o_hbm)

    return gather_kernel(W, idx).reshape(B, H, D)


# speedup vs baseline: 1.8714x; 1.0727x over previous
import jax
import jax.numpy as jnp
from jax.experimental import pallas as pl
from jax.experimental.pallas import tpu as pltpu
from jax.experimental.pallas import tpu_sc as plsc

_WIN = 128
_BLK = 512


def kernel(x, W):
    B, H = x.shape
    V, D = W.shape
    N = B * H

    idx = x.reshape(1, N)
    mesh = plsc.VectorSubcoreMesh(core_axis_name="core",
                                  subcore_axis_name="subcore")

    @pl.kernel(out_type=jax.ShapeDtypeStruct((N, D), W.dtype), mesh=mesh,
               compiler_params=pltpu.CompilerParams(use_tc_tiling_on_sc=False),
               scratch_types=[pltpu.SemaphoreType.DMA])
    def gather_kernel(w_hbm, i_hbm, o_hbm, sem):
        def body(i_vmem, o_vmem):
            copies = [
                pltpu.async_copy(
                    w_hbm.at[i_vmem.at[0, pl.ds(j * _WIN, _WIN)]],
                    o_vmem.at[pl.ds(j * _WIN, _WIN)],
                    sem,
                )
                for j in range(_BLK // _WIN)
            ]
            for c in copies:
                c.wait()

        pltpu.emit_pipeline(
            body,
            grid=(N // _BLK,),
            in_specs=[pl.BlockSpec((1, _BLK), index_map=lambda i: (0, i))],
            out_specs=[pl.BlockSpec((_BLK, D), index_map=lambda i: (i, 0))],
            core_axis_name=("core", "subcore"),
            dimension_semantics=(pltpu.PARALLEL,),
        )(i_hbm, o_hbm)

    return gather_kernel(W, idx).reshape(B, H, D)
